# baseline (speedup 1.0000x reference)
import jax
import jax.numpy as jnp
from jax.experimental import pallas as pl
from jax.experimental.pallas import tpu as pltpu
from jax.experimental.pallas import tpu_sc as plsc

D_MODEL = 64
GATHER_WINDOW = 128
LANES = 16
SCALE = 8.0


def _emb_body(table_hbm, idx_hbm, out_hbm):
    num_idx = idx_hbm.shape[1]

    def body(i_vmem, o_vmem):
        pltpu.sync_copy(table_hbm.at[i_vmem.at[0]], o_vmem)

        @pl.loop(0, GATHER_WINDOW)
        def _(r):
            for c in range(0, D_MODEL, LANES):
                o_vmem[r, pl.ds(c, LANES)] = (
                    o_vmem[r, pl.ds(c, LANES)] * SCALE
                )

    pltpu.emit_pipeline(
        body,
        grid=(num_idx // GATHER_WINDOW,),
        in_specs=[pl.BlockSpec((1, GATHER_WINDOW), index_map=lambda i: (0, i))],
        out_specs=[pl.BlockSpec((GATHER_WINDOW, D_MODEL),
                                index_map=lambda i: (i, 0))],
        core_axis_name=("c", "s"),
        dimension_semantics=(pltpu.PARALLEL,),
    )(idx_hbm, out_hbm)


def kernel(x, table):
    batch_shape = x.shape
    idx = x.reshape(1, -1).astype(jnp.int32)
    n = idx.shape[1]
    mesh = plsc.VectorSubcoreMesh(core_axis_name="c", subcore_axis_name="s")
    gathered = pl.kernel(
        _emb_body,
        out_type=jax.ShapeDtypeStruct((n, D_MODEL), table.dtype),
        mesh=mesh,
        compiler_params=pltpu.CompilerParams(use_tc_tiling_on_sc=False),
    )(table, idx)
    return gathered.reshape(*batch_shape, D_MODEL)


# speedup vs baseline: 1.2746x; 1.2746x over previous
import jax
import jax.numpy as jnp
from jax.experimental import pallas as pl
from jax.experimental.pallas import tpu as pltpu
from jax.experimental.pallas import tpu_sc as plsc

D_MODEL = 64
GW = 128
LANES = 16
SCALE = 8.0
NBUF = 4
NW = 32
ROW_UNROLL = 8


def _emb_body(table_hbm, idx_hbm, out_hbm, idx_v, rows_g, rows_w, gsem, wsem):
    steps = idx_hbm.shape[1]
    wid = jax.lax.axis_index("c") * 16 + jax.lax.axis_index("s")
    base_row = wid * (steps * GW)

    pltpu.sync_copy(idx_hbm.at[wid], idx_v)

    def gather(j, b):
        return pltpu.make_async_copy(
            table_hbm.at[idx_v.at[j]], rows_g.at[b], gsem.at[b])

    def writeback(j, b):
        return pltpu.make_async_copy(
            rows_w.at[b], out_hbm.at[pl.ds(base_row + j * GW, GW)],
            wsem.at[b])

    def scale(b):
        @pl.loop(0, GW, step=ROW_UNROLL)
        def _(r):
            for rr in range(ROW_UNROLL):
                for c in range(0, D_MODEL, LANES):
                    rows_w[b, r + rr, pl.ds(c, LANES)] = (
                        rows_g[b, r + rr, pl.ds(c, LANES)] * SCALE
                    )

    for b in range(NBUF):
        gather(b, b).start()

    for b in range(NBUF):
        gather(b, b).wait()
        scale(b)
        gather(b + NBUF, b).start()
        writeback(b, b).start()

    @pl.loop(NBUF, steps - NBUF, step=NBUF)
    def _(g):
        for b in range(NBUF):
            j = g + b
            gather(j, b).wait()
            writeback(j - NBUF, b).wait()
            scale(b)
            gather(j + NBUF, b).start()
            writeback(j, b).start()

    for b in range(NBUF):
        j = steps - NBUF + b
        gather(j, b).wait()
        writeback(j - NBUF, b).wait()
        scale(b)
        writeback(j, b).start()

    for b in range(NBUF):
        writeback(steps - NBUF + b, b).wait()


def kernel(x, table):
    batch_shape = x.shape
    idx = x.reshape(-1).astype(jnp.int32)
    n = idx.shape[0]
    steps = n // (NW * GW)
    idx3 = idx.reshape(NW, steps, GW)
    mesh = plsc.VectorSubcoreMesh(core_axis_name="c", subcore_axis_name="s")
    gathered = pl.kernel(
        _emb_body,
        out_type=jax.ShapeDtypeStruct((n, D_MODEL), table.dtype),
        mesh=mesh,
        scratch_types=[
            pltpu.VMEM((steps, GW), jnp.int32),
            pltpu.VMEM((NBUF, GW, D_MODEL), jnp.float32),
            pltpu.VMEM((NBUF, GW, D_MODEL), jnp.float32),
            pltpu.SemaphoreType.DMA((NBUF,)),
            pltpu.SemaphoreType.DMA((NBUF,)),
        ],
        compiler_params=pltpu.CompilerParams(use_tc_tiling_on_sc=False),
    )(table, idx3)
    return gathered.reshape(*batch_shape, D_MODEL)
